# baseline (speedup 1.0000x reference)
import jax
import jax.numpy as jnp
import numpy as np
from jax.experimental import pallas as pl
from jax.experimental.pallas import tpu as pltpu

B = 8
S = 1024
H = 2048
HQ = 16
HKV = 4
D = 128
G = HQ // HKV
NQ = HQ * D
NKV = HKV * D

BM = 1024
BQ = 256
NCHUNK = S // BQ


def _qkv_attn_kernel(x_ref, w_ref, b_ref, o_ref, q_scr, k_scr, v_scr):
    n = pl.program_id(1)

    @pl.when(n == 0)
    def _qkv():
        x = x_ref[...]
        for c in range(HKV):
            sl = slice(c * G * D, (c + 1) * G * D)
            q_scr[c] = (jnp.dot(x, w_ref[:, sl], preferred_element_type=jnp.float32)
                        + b_ref[:, sl]).astype(jnp.bfloat16)
        ky = (jnp.dot(x, w_ref[:, NQ:NQ + NKV], preferred_element_type=jnp.float32)
              + b_ref[:, NQ:NQ + NKV]).astype(jnp.bfloat16)
        vy = (jnp.dot(x, w_ref[:, NQ + NKV:], preferred_element_type=jnp.float32)
              + b_ref[:, NQ + NKV:]).astype(jnp.bfloat16)
        for c in range(HKV):
            k_scr[c] = ky[:, c * D:(c + 1) * D]
            v_scr[c] = vy[:, c * D:(c + 1) * D]

    @pl.when(n > 0)
    def _attn():
        h = n - 1
        k = k_scr[h]
        v = v_scr[h]
        row = jax.lax.broadcasted_iota(jnp.int32, (BQ, BQ), 0)
        col = jax.lax.broadcasted_iota(jnp.int32, (BQ, BQ), 1)
        dmask = row >= col
        for g in range(G):
            for i in range(NCHUNK):
                ext = (i + 1) * BQ
                q_g = q_scr[h, i * BQ:(i + 1) * BQ, g * D:(g + 1) * D]
                s = jax.lax.dot_general(q_g, k[:ext, :], (((1,), (1,)), ((), ())),
                                        preferred_element_type=jnp.float32)
                m = jnp.max(s, axis=-1, keepdims=True)
                e = jnp.exp2(s - m)
                e_diag = jnp.where(dmask, e[:, i * BQ:], jnp.float32(0.0))
                denom = jnp.sum(e_diag, axis=-1, keepdims=True)
                o_g = jnp.dot(e_diag.astype(jnp.bfloat16), v[i * BQ:ext, :],
                              preferred_element_type=jnp.float32)
                if i > 0:
                    e_main = e[:, :i * BQ]
                    denom = denom + jnp.sum(e_main, axis=-1, keepdims=True)
                    o_g = o_g + jnp.dot(e_main.astype(jnp.bfloat16), v[:i * BQ, :],
                                        preferred_element_type=jnp.float32)
                o_g = o_g * (1.0 / denom)
                o_ref[i * BQ:(i + 1) * BQ, g * D:(g + 1) * D] = o_g.astype(jnp.bfloat16)


def _out_kernel(x_ref, w_ref, b_ref, o_ref):
    o_ref[...] = (jnp.dot(x_ref[...], w_ref[...],
                          preferred_element_type=jnp.float32) + b_ref[...])


def kernel(x, Wq, bq, Wk, bk, Wv, bv, Wo, bo, b_start_loc, b_seq_len, max_seq_len):
    T = x.shape[0]
    scale = (1.0 / np.sqrt(D)) * np.log2(np.e)
    Wqkv = jnp.concatenate([Wq * scale, Wk, Wv], axis=1).astype(jnp.bfloat16)
    bqkv = jnp.concatenate([bq * scale, bk, bv])[None, :]
    Nqkv = NQ + 2 * NKV

    x16 = x.astype(jnp.bfloat16)
    o = pl.pallas_call(
        _qkv_attn_kernel,
        grid=(B, 1 + HKV),
        in_specs=[
            pl.BlockSpec((S, H), lambda b, n: (b, 0)),
            pl.BlockSpec((H, Nqkv), lambda b, n: (0, 0)),
            pl.BlockSpec((1, Nqkv), lambda b, n: (0, 0)),
        ],
        out_specs=pl.BlockSpec(
            (S, G * D),
            lambda b, n: (b, jnp.maximum(n - 1, 0))),
        out_shape=jax.ShapeDtypeStruct((T, NQ), jnp.bfloat16),
        scratch_shapes=[
            pltpu.VMEM((HKV, S, G * D), jnp.bfloat16),
            pltpu.VMEM((HKV, S, D), jnp.bfloat16),
            pltpu.VMEM((HKV, S, D), jnp.bfloat16),
        ],
        compiler_params=pltpu.CompilerParams(
            dimension_semantics=(pltpu.PARALLEL, pltpu.ARBITRARY),
            vmem_limit_bytes=100 * 1024 * 1024,
        ),
    )(x16, Wqkv, bqkv)

    out = pl.pallas_call(
        _out_kernel,
        grid=(T // BM,),
        in_specs=[
            pl.BlockSpec((BM, NQ), lambda i: (i, 0)),
            pl.BlockSpec((NQ, H), lambda i: (0, 0)),
            pl.BlockSpec((1, H), lambda i: (0, 0)),
        ],
        out_specs=pl.BlockSpec((BM, H), lambda i: (i, 0)),
        out_shape=jax.ShapeDtypeStruct((T, H), jnp.float32),
        compiler_params=pltpu.CompilerParams(
            dimension_semantics=(pltpu.PARALLEL,),
            vmem_limit_bytes=100 * 1024 * 1024,
        ),
    )(o, Wo.astype(jnp.bfloat16), bo[None, :])

    return out


# speedup vs baseline: 1.0153x; 1.0153x over previous
import jax
import jax.numpy as jnp
import numpy as np
from jax.experimental import pallas as pl
from jax.experimental.pallas import tpu as pltpu

B = 8
S = 1024
H = 2048
HQ = 16
HKV = 4
D = 128
G = HQ // HKV
NQ = HQ * D
NKV = HKV * D

BM = 512
BQ = 256
NCHUNK = S // BQ


def _qkv_kernel(x_ref, w_ref, b_ref, q_ref, k_ref, v_ref):
    x = x_ref[...].astype(jnp.bfloat16)
    y = jnp.dot(x, w_ref[...], preferred_element_type=jnp.float32) + b_ref[...]
    y = y.astype(jnp.bfloat16)
    q_ref[...] = y[:, :NQ]
    k_ref[...] = y[:, NQ:NQ + NKV]
    v_ref[...] = y[:, NQ + NKV:]


def _attn_kernel(q_ref, k_ref, v_ref, o_ref):
    k = k_ref[...]
    v = v_ref[...]
    row = jax.lax.broadcasted_iota(jnp.int32, (BQ, BQ), 0)
    col = jax.lax.broadcasted_iota(jnp.int32, (BQ, BQ), 1)
    dmask = row >= col
    def _one_g(g, _):
        for i in range(NCHUNK):
            ext = (i + 1) * BQ
            c0 = pl.multiple_of(g * D, D)
            q_g = q_ref[i * BQ:(i + 1) * BQ, pl.ds(c0, D)]
            s = jax.lax.dot_general(q_g, k[:ext, :], (((1,), (1,)), ((), ())),
                                    preferred_element_type=jnp.float32)
            m = jnp.max(s, axis=-1, keepdims=True)
            e = jnp.exp2(s - m)
            e_diag = jnp.where(dmask, e[:, i * BQ:], jnp.float32(0.0))
            denom = jnp.sum(e_diag, axis=-1, keepdims=True)
            o_g = jnp.dot(e_diag.astype(jnp.bfloat16), v[i * BQ:ext, :],
                          preferred_element_type=jnp.float32)
            if i > 0:
                e_main = e[:, :i * BQ]
                denom = denom + jnp.sum(e_main, axis=-1, keepdims=True)
                o_g = o_g + jnp.dot(e_main.astype(jnp.bfloat16), v[:i * BQ, :],
                                    preferred_element_type=jnp.float32)
            o_g = o_g * (1.0 / denom)
            o_ref[i * BQ:(i + 1) * BQ, pl.ds(c0, D)] = o_g.astype(jnp.bfloat16)
        return _

    jax.lax.fori_loop(0, G, _one_g, 0)


def _out_kernel(x_ref, w_ref, b_ref, o_ref):
    o_ref[...] = (jnp.dot(x_ref[...], w_ref[...],
                          preferred_element_type=jnp.float32) + b_ref[...])


def kernel(x, Wq, bq, Wk, bk, Wv, bv, Wo, bo, b_start_loc, b_seq_len, max_seq_len):
    T = x.shape[0]
    scale = (1.0 / np.sqrt(D)) * np.log2(np.e)
    Wqkv = jnp.concatenate([Wq * scale, Wk, Wv], axis=1).astype(jnp.bfloat16)
    bqkv = jnp.concatenate([bq * scale, bk, bv])[None, :]
    Nqkv = NQ + 2 * NKV

    q, k, v = pl.pallas_call(
        _qkv_kernel,
        grid=(T // BM,),
        in_specs=[
            pl.BlockSpec((BM, H), lambda i: (i, 0)),
            pl.BlockSpec((H, Nqkv), lambda i: (0, 0)),
            pl.BlockSpec((1, Nqkv), lambda i: (0, 0)),
        ],
        out_specs=[
            pl.BlockSpec((BM, NQ), lambda i: (i, 0)),
            pl.BlockSpec((BM, NKV), lambda i: (i, 0)),
            pl.BlockSpec((BM, NKV), lambda i: (i, 0)),
        ],
        out_shape=[
            jax.ShapeDtypeStruct((T, NQ), jnp.bfloat16),
            jax.ShapeDtypeStruct((T, NKV), jnp.bfloat16),
            jax.ShapeDtypeStruct((T, NKV), jnp.bfloat16),
        ],
        compiler_params=pltpu.CompilerParams(
            dimension_semantics=(pltpu.PARALLEL,),
            vmem_limit_bytes=100 * 1024 * 1024,
        ),
    )(x, Wqkv, bqkv)

    o = pl.pallas_call(
        _attn_kernel,
        grid=(B, HKV),
        in_specs=[
            pl.BlockSpec((S, G * D), lambda b, h: (b, h)),
            pl.BlockSpec((S, D), lambda b, h: (b, h)),
            pl.BlockSpec((S, D), lambda b, h: (b, h)),
        ],
        out_specs=pl.BlockSpec((S, G * D), lambda b, h: (b, h)),
        out_shape=jax.ShapeDtypeStruct((T, NQ), jnp.bfloat16),
        compiler_params=pltpu.CompilerParams(
            dimension_semantics=(pltpu.PARALLEL, pltpu.ARBITRARY),
            vmem_limit_bytes=100 * 1024 * 1024,
        ),
    )(q, k, v)

    out = pl.pallas_call(
        _out_kernel,
        grid=(T // BM,),
        in_specs=[
            pl.BlockSpec((BM, NQ), lambda i: (i, 0)),
            pl.BlockSpec((NQ, H), lambda i: (0, 0)),
            pl.BlockSpec((1, H), lambda i: (0, 0)),
        ],
        out_specs=pl.BlockSpec((BM, H), lambda i: (i, 0)),
        out_shape=jax.ShapeDtypeStruct((T, H), jnp.float32),
        compiler_params=pltpu.CompilerParams(
            dimension_semantics=(pltpu.PARALLEL,),
            vmem_limit_bytes=100 * 1024 * 1024,
        ),
    )(o, Wo.astype(jnp.bfloat16), bo[None, :])

    return out


# speedup vs baseline: 1.2029x; 1.1849x over previous
import jax
import jax.numpy as jnp
import numpy as np
from jax.experimental import pallas as pl
from jax.experimental.pallas import tpu as pltpu

B = 8
S = 1024
H = 2048
HQ = 16
HKV = 4
D = 128
G = HQ // HKV
NQ = HQ * D
NKV = HKV * D

BM = 1024
BQ = 512
NCHUNK = S // BQ


def _qkv_kernel(x_ref, w_ref, b_ref, q_ref, k_ref, v_ref):
    x = x_ref[...].astype(jnp.bfloat16)
    y = jnp.dot(x, w_ref[...], preferred_element_type=jnp.float32) + b_ref[...]
    y = y.astype(jnp.bfloat16)
    q_ref[...] = y[:, :NQ]
    k_ref[...] = y[:, NQ:NQ + NKV]
    v_ref[...] = y[:, NQ + NKV:]


def _attn_out_kernel(q_ref, k_ref, v_ref, wo_ref, bo_ref, out_ref, o_scr):
    h = pl.program_id(1)
    k = k_ref[...]
    va = jnp.concatenate(
        [v_ref[...], jnp.ones((S, 16), jnp.bfloat16)], axis=1)
    row = jax.lax.broadcasted_iota(jnp.int32, (BQ, BQ), 0)
    col = jax.lax.broadcasted_iota(jnp.int32, (BQ, BQ), 1)
    dmask = row >= col
    for g in range(G):
        for i in range(NCHUNK):
            ext = (i + 1) * BQ
            q_g = q_ref[i * BQ:(i + 1) * BQ, g * D:(g + 1) * D]
            s = jax.lax.dot_general(q_g, k[:ext, :], (((1,), (1,)), ((), ())),
                                    preferred_element_type=jnp.float32)
            m = jnp.max(s, axis=-1, keepdims=True)
            e = jnp.exp2(s - m)
            e_diag = jnp.where(dmask, e[:, i * BQ:], jnp.float32(0.0))
            o_g = jnp.dot(e_diag.astype(jnp.bfloat16), va[i * BQ:ext, :],
                          preferred_element_type=jnp.float32)
            if i > 0:
                e_main = e[:, :i * BQ]
                o_g = o_g + jnp.dot(e_main.astype(jnp.bfloat16), va[:i * BQ, :],
                                    preferred_element_type=jnp.float32)
            o_g = o_g[:, :D] * (1.0 / o_g[:, D:D + 1])
            o_scr[i * BQ:(i + 1) * BQ, g * D:(g + 1) * D] = o_g.astype(jnp.bfloat16)

    proj = jnp.dot(o_scr[...], wo_ref[...], preferred_element_type=jnp.float32)

    @pl.when(h == 0)
    def _():
        out_ref[...] = proj + bo_ref[...]

    @pl.when(h > 0)
    def _():
        out_ref[...] = proj + out_ref[...]


def kernel(x, Wq, bq, Wk, bk, Wv, bv, Wo, bo, b_start_loc, b_seq_len, max_seq_len):
    T = x.shape[0]
    scale = (1.0 / np.sqrt(D)) * np.log2(np.e)
    Wqkv = jnp.concatenate([Wq * scale, Wk, Wv], axis=1).astype(jnp.bfloat16)
    bqkv = jnp.concatenate([bq * scale, bk, bv])[None, :]
    Nqkv = NQ + 2 * NKV

    q, k, v = pl.pallas_call(
        _qkv_kernel,
        grid=(T // BM,),
        in_specs=[
            pl.BlockSpec((BM, H), lambda i: (i, 0)),
            pl.BlockSpec((H, Nqkv), lambda i: (0, 0)),
            pl.BlockSpec((1, Nqkv), lambda i: (0, 0)),
        ],
        out_specs=[
            pl.BlockSpec((BM, NQ), lambda i: (i, 0)),
            pl.BlockSpec((BM, NKV), lambda i: (i, 0)),
            pl.BlockSpec((BM, NKV), lambda i: (i, 0)),
        ],
        out_shape=[
            jax.ShapeDtypeStruct((T, NQ), jnp.bfloat16),
            jax.ShapeDtypeStruct((T, NKV), jnp.bfloat16),
            jax.ShapeDtypeStruct((T, NKV), jnp.bfloat16),
        ],
        compiler_params=pltpu.CompilerParams(
            dimension_semantics=(pltpu.PARALLEL,),
            vmem_limit_bytes=100 * 1024 * 1024,
        ),
    )(x, Wqkv, bqkv)

    out = pl.pallas_call(
        _attn_out_kernel,
        grid=(B, HKV),
        in_specs=[
            pl.BlockSpec((S, G * D), lambda b, h: (b, h)),
            pl.BlockSpec((S, D), lambda b, h: (b, h)),
            pl.BlockSpec((S, D), lambda b, h: (b, h)),
            pl.BlockSpec((G * D, H), lambda b, h: (h, 0)),
            pl.BlockSpec((1, H), lambda b, h: (0, 0)),
        ],
        out_specs=pl.BlockSpec((S, H), lambda b, h: (b, 0)),
        out_shape=jax.ShapeDtypeStruct((T, H), jnp.float32),
        scratch_shapes=[pltpu.VMEM((S, G * D), jnp.bfloat16)],
        compiler_params=pltpu.CompilerParams(
            dimension_semantics=(pltpu.PARALLEL, pltpu.ARBITRARY),
            vmem_limit_bytes=100 * 1024 * 1024,
        ),
    )(q, k, v, Wo.astype(jnp.bfloat16), bo[None, :])

    return out


# speedup vs baseline: 1.2673x; 1.0535x over previous
import jax
import jax.numpy as jnp
import numpy as np
from jax.experimental import pallas as pl
from jax.experimental.pallas import tpu as pltpu

B = 8
S = 1024
H = 2048
HQ = 16
HKV = 4
D = 128
G = HQ // HKV
NQ = HQ * D
NKV = HKV * D

BM = 1024
BQ = 512
NCHUNK = S // BQ


def _qkv_kernel(x_ref, w_ref, b_ref, q_ref, k_ref, v_ref):
    x = x_ref[...].astype(jnp.bfloat16)
    y = jnp.dot(x, w_ref[...], preferred_element_type=jnp.float32) + b_ref[...]
    y = y.astype(jnp.bfloat16)
    q_ref[...] = y[:, :NQ]
    k_ref[...] = y[:, NQ:NQ + NKV]
    v_ref[...] = y[:, NQ + NKV:]


def _attn_kernel(q_ref, k_ref, v_ref, o_ref):
    k = k_ref[...]
    va = jnp.concatenate(
        [v_ref[...], jnp.ones((S, 16), jnp.bfloat16)], axis=1)
    dmasks = {}
    for bq in (512, 256):
        row = jax.lax.broadcasted_iota(jnp.int32, (bq, bq), 0)
        col = jax.lax.broadcasted_iota(jnp.int32, (bq, bq), 1)
        dmasks[bq] = row >= col
    for g in range(G):
        for (r0, bq) in ((0, 512), (512, 256), (768, 256)):
            q_g = q_ref[r0:r0 + bq, g * D:(g + 1) * D]
            s_diag = jax.lax.dot_general(q_g, k[r0:r0 + bq, :],
                                         (((1,), (1,)), ((), ())),
                                         preferred_element_type=jnp.float32)
            m = jnp.max(s_diag, axis=-1, keepdims=True)
            if r0 > 0:
                s_main = jax.lax.dot_general(q_g, k[:r0, :],
                                             (((1,), (1,)), ((), ())),
                                             preferred_element_type=jnp.float32)
                m = jnp.maximum(m, jnp.max(s_main, axis=-1, keepdims=True))
            e_diag = jnp.where(dmasks[bq], jnp.exp2(s_diag - m), jnp.float32(0.0))
            o_g = jnp.dot(e_diag.astype(jnp.bfloat16), va[r0:r0 + bq, :],
                          preferred_element_type=jnp.float32)
            if r0 > 0:
                e_main = jnp.exp2(s_main - m)
                o_g = o_g + jnp.dot(e_main.astype(jnp.bfloat16), va[:r0, :],
                                    preferred_element_type=jnp.float32)
            o_g = o_g[:, :D] * (1.0 / o_g[:, D:D + 1])
            o_ref[r0:r0 + bq, g * D:(g + 1) * D] = o_g.astype(jnp.bfloat16)


def _out_kernel(x_ref, w_ref, b_ref, o_ref):
    o_ref[...] = (jnp.dot(x_ref[...], w_ref[...],
                          preferred_element_type=jnp.float32) + b_ref[...])


def kernel(x, Wq, bq, Wk, bk, Wv, bv, Wo, bo, b_start_loc, b_seq_len, max_seq_len):
    T = x.shape[0]
    scale = (1.0 / np.sqrt(D)) * np.log2(np.e)
    Wqkv = jnp.concatenate([Wq * scale, Wk, Wv], axis=1).astype(jnp.bfloat16)
    bqkv = jnp.concatenate([bq * scale, bk, bv])[None, :]
    Nqkv = NQ + 2 * NKV

    q, k, v = pl.pallas_call(
        _qkv_kernel,
        grid=(T // BM,),
        in_specs=[
            pl.BlockSpec((BM, H), lambda i: (i, 0)),
            pl.BlockSpec((H, Nqkv), lambda i: (0, 0)),
            pl.BlockSpec((1, Nqkv), lambda i: (0, 0)),
        ],
        out_specs=[
            pl.BlockSpec((BM, NQ), lambda i: (i, 0)),
            pl.BlockSpec((BM, NKV), lambda i: (i, 0)),
            pl.BlockSpec((BM, NKV), lambda i: (i, 0)),
        ],
        out_shape=[
            jax.ShapeDtypeStruct((T, NQ), jnp.bfloat16),
            jax.ShapeDtypeStruct((T, NKV), jnp.bfloat16),
            jax.ShapeDtypeStruct((T, NKV), jnp.bfloat16),
        ],
        compiler_params=pltpu.CompilerParams(
            dimension_semantics=(pltpu.PARALLEL,),
            vmem_limit_bytes=100 * 1024 * 1024,
        ),
    )(x, Wqkv, bqkv)

    o = pl.pallas_call(
        _attn_kernel,
        grid=(B, HKV),
        in_specs=[
            pl.BlockSpec((S, G * D), lambda b, h: (b, h)),
            pl.BlockSpec((S, D), lambda b, h: (b, h)),
            pl.BlockSpec((S, D), lambda b, h: (b, h)),
        ],
        out_specs=pl.BlockSpec((S, G * D), lambda b, h: (b, h)),
        out_shape=jax.ShapeDtypeStruct((T, NQ), jnp.bfloat16),
        compiler_params=pltpu.CompilerParams(
            dimension_semantics=(pltpu.PARALLEL, pltpu.ARBITRARY),
            vmem_limit_bytes=100 * 1024 * 1024,
        ),
    )(q, k, v)

    out = pl.pallas_call(
        _out_kernel,
        grid=(T // BM,),
        in_specs=[
            pl.BlockSpec((BM, NQ), lambda i: (i, 0)),
            pl.BlockSpec((NQ, H), lambda i: (0, 0)),
            pl.BlockSpec((1, H), lambda i: (0, 0)),
        ],
        out_specs=pl.BlockSpec((BM, H), lambda i: (i, 0)),
        out_shape=jax.ShapeDtypeStruct((T, H), jnp.float32),
        compiler_params=pltpu.CompilerParams(
            dimension_semantics=(pltpu.PARALLEL,),
            vmem_limit_bytes=100 * 1024 * 1024,
        ),
    )(o, Wo.astype(jnp.bfloat16), bo[None, :])

    return out


# speedup vs baseline: 1.3195x; 1.0412x over previous
import jax
import jax.numpy as jnp
import numpy as np
from jax.experimental import pallas as pl
from jax.experimental.pallas import tpu as pltpu

B = 8
S = 1024
H = 2048
HQ = 16
HKV = 4
D = 128
G = HQ // HKV
NQ = HQ * D
NKV = HKV * D

BM = 1024
BQ = 512
NCHUNK = S // BQ


def _qkv_kernel(x_ref, w_ref, b_ref, q_ref, k_ref, v_ref):
    x = x_ref[...].astype(jnp.bfloat16)
    y = jnp.dot(x, w_ref[...], preferred_element_type=jnp.float32) + b_ref[...]
    y = y.astype(jnp.bfloat16)
    q_ref[...] = y[:, :NQ]
    k_ref[...] = y[:, NQ:NQ + NKV]
    v_ref[...] = y[:, NQ + NKV:]


def _attn_kernel(q_ref, k_ref, v_ref, o_ref):
    k = k_ref[...]
    va = jnp.concatenate(
        [v_ref[...], jnp.ones((S, 16), jnp.bfloat16)], axis=1)
    dmasks = {}
    for bq in (512, 256):
        row = jax.lax.broadcasted_iota(jnp.int32, (bq, bq), 0)
        col = jax.lax.broadcasted_iota(jnp.int32, (bq, bq), 1)
        dmasks[bq] = row >= col
    for (r0, bq) in ((0, 512), (512, 256), (768, 256)):
        for g in range(G):
            q_g = q_ref[r0:r0 + bq, g * D:(g + 1) * D]
            s_diag = jax.lax.dot_general(q_g, k[r0:r0 + bq, :],
                                         (((1,), (1,)), ((), ())),
                                         preferred_element_type=jnp.float32)
            m = jnp.max(s_diag, axis=-1, keepdims=True)
            if r0 > 0:
                s_main = jax.lax.dot_general(q_g, k[:r0, :],
                                             (((1,), (1,)), ((), ())),
                                             preferred_element_type=jnp.float32)
                m = jnp.maximum(m, jnp.max(s_main, axis=-1, keepdims=True))
            e_diag = jnp.where(dmasks[bq], jnp.exp2(s_diag - m), jnp.float32(0.0))
            o_g = jnp.dot(e_diag.astype(jnp.bfloat16), va[r0:r0 + bq, :],
                          preferred_element_type=jnp.float32)
            if r0 > 0:
                e_main = jnp.exp2(s_main - m)
                o_g = o_g + jnp.dot(e_main.astype(jnp.bfloat16), va[:r0, :],
                                    preferred_element_type=jnp.float32)
            o_g = o_g[:, :D] * (1.0 / o_g[:, D:D + 1])
            o_ref[r0:r0 + bq, g * D:(g + 1) * D] = o_g.astype(jnp.bfloat16)


def _out_kernel(x_ref, w_ref, b_ref, o_ref):
    o_ref[...] = (jnp.dot(x_ref[...], w_ref[...],
                          preferred_element_type=jnp.float32) + b_ref[...])


def kernel(x, Wq, bq, Wk, bk, Wv, bv, Wo, bo, b_start_loc, b_seq_len, max_seq_len):
    T = x.shape[0]
    scale = (1.0 / np.sqrt(D)) * np.log2(np.e)
    Wqkv = jnp.concatenate([Wq * scale, Wk, Wv], axis=1).astype(jnp.bfloat16)
    bqkv = jnp.concatenate([bq * scale, bk, bv])[None, :]
    Nqkv = NQ + 2 * NKV

    q, k, v = pl.pallas_call(
        _qkv_kernel,
        grid=(T // BM,),
        in_specs=[
            pl.BlockSpec((BM, H), lambda i: (i, 0)),
            pl.BlockSpec((H, Nqkv), lambda i: (0, 0)),
            pl.BlockSpec((1, Nqkv), lambda i: (0, 0)),
        ],
        out_specs=[
            pl.BlockSpec((BM, NQ), lambda i: (i, 0)),
            pl.BlockSpec((BM, NKV), lambda i: (i, 0)),
            pl.BlockSpec((BM, NKV), lambda i: (i, 0)),
        ],
        out_shape=[
            jax.ShapeDtypeStruct((T, NQ), jnp.bfloat16),
            jax.ShapeDtypeStruct((T, NKV), jnp.bfloat16),
            jax.ShapeDtypeStruct((T, NKV), jnp.bfloat16),
        ],
        compiler_params=pltpu.CompilerParams(
            dimension_semantics=(pltpu.PARALLEL,),
            vmem_limit_bytes=100 * 1024 * 1024,
        ),
    )(x, Wqkv, bqkv)

    o = pl.pallas_call(
        _attn_kernel,
        grid=(B, HKV),
        in_specs=[
            pl.BlockSpec((S, G * D), lambda b, h: (b, h)),
            pl.BlockSpec((S, D), lambda b, h: (b, h)),
            pl.BlockSpec((S, D), lambda b, h: (b, h)),
        ],
        out_specs=pl.BlockSpec((S, G * D), lambda b, h: (b, h)),
        out_shape=jax.ShapeDtypeStruct((T, NQ), jnp.bfloat16),
        compiler_params=pltpu.CompilerParams(
            dimension_semantics=(pltpu.PARALLEL, pltpu.ARBITRARY),
            vmem_limit_bytes=100 * 1024 * 1024,
        ),
    )(q, k, v)

    out = pl.pallas_call(
        _out_kernel,
        grid=(T // BM,),
        in_specs=[
            pl.BlockSpec((BM, NQ), lambda i: (i, 0)),
            pl.BlockSpec((NQ, H), lambda i: (0, 0)),
            pl.BlockSpec((1, H), lambda i: (0, 0)),
        ],
        out_specs=pl.BlockSpec((BM, H), lambda i: (i, 0)),
        out_shape=jax.ShapeDtypeStruct((T, H), jnp.float32),
        compiler_params=pltpu.CompilerParams(
            dimension_semantics=(pltpu.PARALLEL,),
            vmem_limit_bytes=100 * 1024 * 1024,
        ),
    )(o, Wo.astype(jnp.bfloat16), bo[None, :])

    return out


# speedup vs baseline: 1.3205x; 1.0008x over previous
import jax
import jax.numpy as jnp
import numpy as np
from jax.experimental import pallas as pl
from jax.experimental.pallas import tpu as pltpu

B = 8
S = 1024
H = 2048
HQ = 16
HKV = 4
D = 128
G = HQ // HKV
NQ = HQ * D
NKV = HKV * D

BM = 1024
CHUNKS = ((0, 512), (512, 256), (768, 256))


def _qkv_kernel(x_ref, w_ref, b_ref, q_ref, k_ref, v_ref):
    x = x_ref[...].astype(jnp.bfloat16)
    y = jnp.dot(x, w_ref[...], preferred_element_type=jnp.float32) + b_ref[...]
    y = y.astype(jnp.bfloat16)
    q_ref[...] = y[:, :NQ]
    k_ref[...] = y[:, NQ:NQ + NKV]
    v_ref[...] = y[:, NQ + NKV:]


def _attn_kernel(q_ref, k_ref, v_ref, o_ref):
    k = k_ref[...]
    va = jnp.concatenate(
        [v_ref[...], jnp.ones((S, 16), jnp.bfloat16)], axis=1)
    dmasks = {}
    for bq in (512, 256):
        row = jax.lax.broadcasted_iota(jnp.int32, (bq, bq), 0)
        col = jax.lax.broadcasted_iota(jnp.int32, (bq, bq), 1)
        dmasks[bq] = row >= col
    for (r0, bq) in CHUNKS:
        for g in range(G):
            q_g = q_ref[r0:r0 + bq, g * D:(g + 1) * D]
            s_diag = jax.lax.dot_general(q_g, k[r0:r0 + bq, :],
                                         (((1,), (1,)), ((), ())),
                                         preferred_element_type=jnp.float32)
            m = jnp.max(s_diag, axis=-1, keepdims=True)
            if r0 > 0:
                s_main = jax.lax.dot_general(q_g, k[:r0, :],
                                             (((1,), (1,)), ((), ())),
                                             preferred_element_type=jnp.float32)
                m = jnp.maximum(m, jnp.max(s_main, axis=-1, keepdims=True))
            e_diag = jnp.where(dmasks[bq], jnp.exp2(s_diag - m), jnp.float32(0.0))
            o_g = jnp.dot(e_diag.astype(jnp.bfloat16), va[r0:r0 + bq, :],
                          preferred_element_type=jnp.float32)
            if r0 > 0:
                e_main = jnp.exp2(s_main - m)
                o_g = o_g + jnp.dot(e_main.astype(jnp.bfloat16), va[:r0, :],
                                    preferred_element_type=jnp.float32)
            o_g = o_g[:, :D] * (1.0 / o_g[:, D:D + 1])
            o_ref[r0:r0 + bq, g * D:(g + 1) * D] = o_g.astype(jnp.bfloat16)


def _out_kernel(x_ref, w_ref, b_ref, o_ref):
    o_ref[...] = (jnp.dot(x_ref[...], w_ref[...],
                          preferred_element_type=jnp.float32) + b_ref[...])


def kernel(x, Wq, bq, Wk, bk, Wv, bv, Wo, bo, b_start_loc, b_seq_len, max_seq_len):
    T = x.shape[0]
    scale = (1.0 / np.sqrt(D)) * np.log2(np.e)
    Wqkv = jnp.concatenate([Wq * scale, Wk, Wv], axis=1).astype(jnp.bfloat16)
    bqkv = jnp.concatenate([bq * scale, bk, bv])[None, :]
    Nqkv = NQ + 2 * NKV

    q, k, v = pl.pallas_call(
        _qkv_kernel,
        grid=(T // BM,),
        in_specs=[
            pl.BlockSpec((BM, H), lambda i: (i, 0)),
            pl.BlockSpec((H, Nqkv), lambda i: (0, 0)),
            pl.BlockSpec((1, Nqkv), lambda i: (0, 0)),
        ],
        out_specs=[
            pl.BlockSpec((BM, NQ), lambda i: (i, 0)),
            pl.BlockSpec((BM, NKV), lambda i: (i, 0)),
            pl.BlockSpec((BM, NKV), lambda i: (i, 0)),
        ],
        out_shape=[
            jax.ShapeDtypeStruct((T, NQ), jnp.bfloat16),
            jax.ShapeDtypeStruct((T, NKV), jnp.bfloat16),
            jax.ShapeDtypeStruct((T, NKV), jnp.bfloat16),
        ],
        compiler_params=pltpu.CompilerParams(
            dimension_semantics=(pltpu.PARALLEL,),
            vmem_limit_bytes=100 * 1024 * 1024,
        ),
    )(x, Wqkv, bqkv)

    o = pl.pallas_call(
        _attn_kernel,
        grid=(B, HKV),
        in_specs=[
            pl.BlockSpec((S, G * D), lambda b, h: (b, h)),
            pl.BlockSpec((S, D), lambda b, h: (b, h)),
            pl.BlockSpec((S, D), lambda b, h: (b, h)),
        ],
        out_specs=pl.BlockSpec((S, G * D), lambda b, h: (b, h)),
        out_shape=jax.ShapeDtypeStruct((T, NQ), jnp.bfloat16),
        compiler_params=pltpu.CompilerParams(
            dimension_semantics=(pltpu.PARALLEL, pltpu.ARBITRARY),
            vmem_limit_bytes=100 * 1024 * 1024,
        ),
    )(q, k, v)

    out = pl.pallas_call(
        _out_kernel,
        grid=(T // BM,),
        in_specs=[
            pl.BlockSpec((BM, NQ), lambda i: (i, 0)),
            pl.BlockSpec((NQ, H), lambda i: (0, 0)),
            pl.BlockSpec((1, H), lambda i: (0, 0)),
        ],
        out_specs=pl.BlockSpec((BM, H), lambda i: (i, 0)),
        out_shape=jax.ShapeDtypeStruct((T, H), jnp.float32),
        compiler_params=pltpu.CompilerParams(
            dimension_semantics=(pltpu.PARALLEL,),
            vmem_limit_bytes=100 * 1024 * 1024,
        ),
    )(o, Wo.astype(jnp.bfloat16), bo[None, :])

    return out
